# baseline (speedup 1.0000x reference)
import jax
import jax.numpy as jnp
from jax import lax
from jax.experimental import pallas as pl
from jax.experimental.pallas import tpu as pltpu
from jax.experimental.pallas import tpu_sc as plsc

_N = 5000
_MAX_DET = 300
_SIGMA = 0.5
_SCORE_THRESH = 0.001

_L = 16
_NS = 16
_NPAD = 5120
_CHUNK = _NPAD // _NS
_NV = _CHUNK // _L
_OUT_PAD = 320
_MAT = _NS * _L


def _nms_body(x0_hbm, y0_hbm, x1_hbm, y1_hbm, sc_hbm, out_hbm,
              x0_v, y0_v, x1_v, y1_v, area_v, sc_a, sc_b, out_v, cand_v,
              mat_v, shared_v):
    cid = lax.axis_index("c")
    sid = lax.axis_index("s")
    base = sid * _CHUNK

    pltpu.sync_copy(x0_hbm.at[pl.ds(base, _CHUNK)], x0_v)
    pltpu.sync_copy(y0_hbm.at[pl.ds(base, _CHUNK)], y0_v)
    pltpu.sync_copy(x1_hbm.at[pl.ds(base, _CHUNK)], x1_v)
    pltpu.sync_copy(y1_hbm.at[pl.ds(base, _CHUNK)], y1_v)
    pltpu.sync_copy(sc_hbm.at[pl.ds(base, _CHUNK)], sc_a)

    lane = lax.iota(jnp.int32, _L)
    lane_f = lane.astype(jnp.float32)
    mask0 = lane == 0

    for j in range(_NV):
        sl = pl.ds(j * _L, _L)
        area_v[sl] = (x1_v[sl] - x0_v[sl]) * (y1_v[sl] - y0_v[sl])

    def publish_cand(m, mj):
        mloc = jnp.max(m)
        flat = mj * jnp.float32(_L) + lane_f
        lidx_f = jnp.min(jnp.where(m == mloc, flat, jnp.float32(99999.0)))
        li_b = jnp.broadcast_to(lidx_f.astype(jnp.int32), (_L,))
        cx0 = plsc.load_gather(x0_v, [li_b])
        cy0 = plsc.load_gather(y0_v, [li_b])
        cx1 = plsc.load_gather(x1_v, [li_b])
        cy1 = plsc.load_gather(y1_v, [li_b])
        row = jnp.broadcast_to(mloc, (_L,))
        row = jnp.where(lane == 1, lidx_f, row)
        row = jnp.where(lane == 2, cx0, row)
        row = jnp.where(lane == 3, cy0, row)
        row = jnp.where(lane == 4, cx1, row)
        row = jnp.where(lane == 5, cy1, row)
        cand_v[...] = row

    m = jnp.full((_L,), -2.0, jnp.float32)
    mj = jnp.zeros((_L,), jnp.float32)
    for j in range(_NV):
        v = sc_a[pl.ds(j * _L, _L)]
        gt = v > m
        mj = jnp.where(gt, jnp.float32(j), mj)
        m = jnp.where(gt, v, m)
    publish_cand(m, mj)

    def substep(i, off, src, dst):
        pltpu.sync_copy(cand_v, shared_v.at[pl.ds(off + sid * _L, _L)])
        plsc.subcore_barrier()
        pltpu.sync_copy(shared_v.at[pl.ds(off, _MAT)], mat_v)

        maxes = plsc.load_gather(mat_v, [lane * _L])
        mg = jnp.max(maxes)
        wt = plsc.all_reduce_ffs(maxes == mg)
        wrow = wt * _L
        lidxw = plsc.load_gather(mat_v, [wrow + 1])
        wx0 = plsc.load_gather(mat_v, [wrow + 2])
        wy0 = plsc.load_gather(mat_v, [wrow + 3])
        wx1 = plsc.load_gather(mat_v, [wrow + 4])
        wy1 = plsc.load_gather(mat_v, [wrow + 5])
        warea = (wx1 - wx0) * (wy1 - wy0)

        mg_t = jnp.where(mg < jnp.float32(_SCORE_THRESH), jnp.float32(0.0),
                         mg)
        plsc.store_scatter(out_v, [jnp.broadcast_to(i, (_L,))],
                           jnp.broadcast_to(mg_t, (_L,)), mask=mask0)

        plsc.store_scatter(src, [lidxw.astype(jnp.int32)],
                           jnp.full((_L,), -1.0, jnp.float32),
                           mask=mask0 & (wt == sid))

        m = jnp.full((_L,), -2.0, jnp.float32)
        mj = jnp.zeros((_L,), jnp.float32)
        for j in range(_NV):
            sl = pl.ds(j * _L, _L)
            iw = jnp.maximum(
                jnp.minimum(wx1, x1_v[sl]) - jnp.maximum(wx0, x0_v[sl]), 0.0)
            ih = jnp.maximum(
                jnp.minimum(wy1, y1_v[sl]) - jnp.maximum(wy0, y0_v[sl]), 0.0)
            inter = iw * ih
            union = warea + area_v[sl] - inter
            iou = inter / (union + jnp.float32(1e-9))
            dec = jnp.exp(-(iou * iou) / jnp.float32(_SIGMA))
            sc = src[sl] * dec
            dst[sl] = sc
            gt = sc > m
            mj = jnp.where(gt, jnp.float32(j), mj)
            m = jnp.where(gt, sc, m)
        publish_cand(m, mj)

    def step(k, carry):
        i = k * 2
        substep(i, 0, sc_a, sc_b)
        substep(i + 1, _MAT, sc_b, sc_a)
        return carry

    lax.fori_loop(0, _MAX_DET // 2, step, 0)

    @pl.when(jnp.logical_and(cid == 0, sid == 0))
    def _():
        pltpu.sync_copy(out_v.at[pl.ds(0, _MAX_DET)], out_hbm)


@jax.jit
def kernel(boxes, scores):
    x0 = jnp.zeros((_NPAD,), jnp.float32).at[:_N].set(boxes[:, 0])
    y0 = jnp.zeros((_NPAD,), jnp.float32).at[:_N].set(boxes[:, 1])
    x1 = jnp.zeros((_NPAD,), jnp.float32).at[:_N].set(boxes[:, 2])
    y1 = jnp.zeros((_NPAD,), jnp.float32).at[:_N].set(boxes[:, 3])
    sc = jnp.full((_NPAD,), -1.0, jnp.float32).at[:_N].set(scores)

    mesh = plsc.VectorSubcoreMesh(core_axis_name="c", subcore_axis_name="s",
                                  num_cores=2, num_subcores=_NS)
    run = pl.kernel(
        _nms_body,
        out_type=jax.ShapeDtypeStruct((_MAX_DET,), jnp.float32),
        mesh=mesh,
        scratch_types=[
            pltpu.VMEM((_CHUNK,), jnp.float32),
            pltpu.VMEM((_CHUNK,), jnp.float32),
            pltpu.VMEM((_CHUNK,), jnp.float32),
            pltpu.VMEM((_CHUNK,), jnp.float32),
            pltpu.VMEM((_CHUNK,), jnp.float32),
            pltpu.VMEM((_CHUNK,), jnp.float32),
            pltpu.VMEM((_CHUNK,), jnp.float32),
            pltpu.VMEM((_OUT_PAD,), jnp.float32),
            pltpu.VMEM((_L,), jnp.float32),
            pltpu.VMEM((_MAT,), jnp.float32),
            pltpu.VMEM_SHARED((2 * _MAT,), jnp.float32),
        ],
        compiler_params=pltpu.CompilerParams(needs_layout_passes=False),
    )
    return run(x0, y0, x1, y1, sc)


# speedup vs baseline: 1.3234x; 1.3234x over previous
import jax
import jax.numpy as jnp
from jax import lax
from jax.experimental import pallas as pl
from jax.experimental.pallas import tpu as pltpu
from jax.experimental.pallas import tpu_sc as plsc

_N = 5000
_MAX_DET = 300
_SIGMA = 0.5
_SCORE_THRESH = 0.001

_L = 16
_NS = 16
_NPAD = 5120
_CHUNK = _NPAD // _NS
_NV = _CHUNK // _L
_OUT_PAD = 320
_MAT = _NS * _L


def _nms_body(x0_hbm, y0_hbm, x1_hbm, y1_hbm, sc_hbm, out_hbm,
              x0_v, y0_v, x1_v, y1_v, area_v, sc_a, sc_b, out_v, cand_v,
              mat_v, shared_v):
    cid = lax.axis_index("c")
    sid = lax.axis_index("s")
    base = sid * _CHUNK

    pltpu.sync_copy(x0_hbm.at[pl.ds(base, _CHUNK)], x0_v)
    pltpu.sync_copy(y0_hbm.at[pl.ds(base, _CHUNK)], y0_v)
    pltpu.sync_copy(x1_hbm.at[pl.ds(base, _CHUNK)], x1_v)
    pltpu.sync_copy(y1_hbm.at[pl.ds(base, _CHUNK)], y1_v)
    pltpu.sync_copy(sc_hbm.at[pl.ds(base, _CHUNK)], sc_a)

    lane = lax.iota(jnp.int32, _L)
    lane_f = lane.astype(jnp.float32)
    mask0 = lane == 0

    for j in range(_NV):
        sl = pl.ds(j * _L, _L)
        area_v[sl] = (x1_v[sl] - x0_v[sl]) * (y1_v[sl] - y0_v[sl])

    def publish_cand(m, mi):
        mloc = jnp.max(m)
        lidx_f = jnp.min(jnp.where(m == mloc, mi, jnp.float32(99999.0)))
        li_b = jnp.broadcast_to(lidx_f.astype(jnp.int32), (_L,))
        cx0 = plsc.load_gather(x0_v, [li_b])
        cy0 = plsc.load_gather(y0_v, [li_b])
        cx1 = plsc.load_gather(x1_v, [li_b])
        cy1 = plsc.load_gather(y1_v, [li_b])
        row = jnp.broadcast_to(mloc, (_L,))
        row = jnp.where(lane == 1, lidx_f, row)
        row = jnp.where(lane == 2, cx0, row)
        row = jnp.where(lane == 3, cy0, row)
        row = jnp.where(lane == 4, cx1, row)
        row = jnp.where(lane == 5, cy1, row)
        cand_v[...] = row

    m = jnp.full((_L,), -2.0, jnp.float32)
    mi = jnp.zeros((_L,), jnp.float32)
    for j in range(_NV):
        v = sc_a[pl.ds(j * _L, _L)]
        gt = v > m
        mi = jnp.where(gt, jnp.float32(j * _L) + lane_f, mi)
        m = jnp.where(gt, v, m)
    publish_cand(m, mi)

    def substep(i, off, src, dst):
        pltpu.sync_copy(cand_v, shared_v.at[pl.ds(off + sid * _L, _L)])
        plsc.subcore_barrier()
        pltpu.sync_copy(shared_v.at[pl.ds(off, _MAT)], mat_v)

        maxes = plsc.load_gather(mat_v, [lane * _L])
        mg = jnp.max(maxes)
        wt = plsc.all_reduce_ffs(maxes == mg)
        wrow = wt * _L
        lidxw = plsc.load_gather(mat_v, [wrow + 1])
        wx0 = plsc.load_gather(mat_v, [wrow + 2])
        wy0 = plsc.load_gather(mat_v, [wrow + 3])
        wx1 = plsc.load_gather(mat_v, [wrow + 4])
        wy1 = plsc.load_gather(mat_v, [wrow + 5])
        warea = (wx1 - wx0) * (wy1 - wy0)

        mg_t = jnp.where(mg < jnp.float32(_SCORE_THRESH), jnp.float32(0.0),
                         mg)
        plsc.store_scatter(out_v, [jnp.broadcast_to(i, (_L,))],
                           jnp.broadcast_to(mg_t, (_L,)), mask=mask0)

        plsc.store_scatter(src, [lidxw.astype(jnp.int32)],
                           jnp.full((_L,), -1.0, jnp.float32),
                           mask=mask0 & (wt == sid))

        m0 = jnp.full((_L,), -2.0, jnp.float32)
        mi0 = jnp.zeros((_L,), jnp.float32)

        @plsc.parallel_loop(0, _CHUNK, _L, unroll=4, carry=(m0, mi0))
        def fused(iv, c):
            m, mi = c
            sl = pl.ds(iv, _L)
            iw = jnp.maximum(
                jnp.minimum(wx1, x1_v[sl]) - jnp.maximum(wx0, x0_v[sl]), 0.0)
            ih = jnp.maximum(
                jnp.minimum(wy1, y1_v[sl]) - jnp.maximum(wy0, y0_v[sl]), 0.0)
            inter = iw * ih
            union = warea + area_v[sl] - inter
            iou = inter / (union + jnp.float32(1e-9))
            dec = jnp.exp(-(iou * iou) / jnp.float32(_SIGMA))
            sc = src[sl] * dec
            dst[sl] = sc
            gt = sc > m
            mi = jnp.where(gt, iv.astype(jnp.float32) + lane_f, mi)
            m = jnp.where(gt, sc, m)
            return (m, mi)

        publish_cand(*fused)

    def step(k, carry):
        i = k * 2
        substep(i, 0, sc_a, sc_b)
        substep(i + 1, _MAT, sc_b, sc_a)
        return carry

    lax.fori_loop(0, _MAX_DET // 2, step, 0)

    @pl.when(jnp.logical_and(cid == 0, sid == 0))
    def _():
        pltpu.sync_copy(out_v.at[pl.ds(0, _MAX_DET)], out_hbm)


@jax.jit
def kernel(boxes, scores):
    x0 = jnp.zeros((_NPAD,), jnp.float32).at[:_N].set(boxes[:, 0])
    y0 = jnp.zeros((_NPAD,), jnp.float32).at[:_N].set(boxes[:, 1])
    x1 = jnp.zeros((_NPAD,), jnp.float32).at[:_N].set(boxes[:, 2])
    y1 = jnp.zeros((_NPAD,), jnp.float32).at[:_N].set(boxes[:, 3])
    sc = jnp.full((_NPAD,), -1.0, jnp.float32).at[:_N].set(scores)

    mesh = plsc.VectorSubcoreMesh(core_axis_name="c", subcore_axis_name="s",
                                  num_cores=2, num_subcores=_NS)
    run = pl.kernel(
        _nms_body,
        out_type=jax.ShapeDtypeStruct((_MAX_DET,), jnp.float32),
        mesh=mesh,
        scratch_types=[
            pltpu.VMEM((_CHUNK,), jnp.float32),
            pltpu.VMEM((_CHUNK,), jnp.float32),
            pltpu.VMEM((_CHUNK,), jnp.float32),
            pltpu.VMEM((_CHUNK,), jnp.float32),
            pltpu.VMEM((_CHUNK,), jnp.float32),
            pltpu.VMEM((_CHUNK,), jnp.float32),
            pltpu.VMEM((_CHUNK,), jnp.float32),
            pltpu.VMEM((_OUT_PAD,), jnp.float32),
            pltpu.VMEM((_L,), jnp.float32),
            pltpu.VMEM((_MAT,), jnp.float32),
            pltpu.VMEM_SHARED((2 * _MAT,), jnp.float32),
        ],
        compiler_params=pltpu.CompilerParams(needs_layout_passes=False),
    )
    return run(x0, y0, x1, y1, sc)


# speedup vs baseline: 1.3301x; 1.0050x over previous
import jax
import jax.numpy as jnp
from jax import lax
from jax.experimental import pallas as pl
from jax.experimental.pallas import tpu as pltpu
from jax.experimental.pallas import tpu_sc as plsc

_N = 5000
_MAX_DET = 300
_SIGMA = 0.5
_SCORE_THRESH = 0.001

_L = 16
_NS = 16
_NPAD = 5120
_CHUNK = _NPAD // _NS
_NV = _CHUNK // _L
_OUT_PAD = 320
_MAT = _NS * _L


def _nms_body(x0_hbm, y0_hbm, x1_hbm, y1_hbm, sc_hbm, out_hbm,
              x0_v, y0_v, x1_v, y1_v, area_v, sc_a, sc_b, out_v, cand_v,
              mat_v, shared_v):
    cid = lax.axis_index("c")
    sid = lax.axis_index("s")
    base = sid * _CHUNK

    pltpu.sync_copy(x0_hbm.at[pl.ds(base, _CHUNK)], x0_v)
    pltpu.sync_copy(y0_hbm.at[pl.ds(base, _CHUNK)], y0_v)
    pltpu.sync_copy(x1_hbm.at[pl.ds(base, _CHUNK)], x1_v)
    pltpu.sync_copy(y1_hbm.at[pl.ds(base, _CHUNK)], y1_v)
    pltpu.sync_copy(sc_hbm.at[pl.ds(base, _CHUNK)], sc_a)

    lane = lax.iota(jnp.int32, _L)
    lane_f = lane.astype(jnp.float32)
    mask0 = lane == 0

    for j in range(_NV):
        sl = pl.ds(j * _L, _L)
        area_v[sl] = (x1_v[sl] - x0_v[sl]) * (y1_v[sl] - y0_v[sl])

    def publish_cand(m, mi):
        mloc = jnp.max(m)
        lidx_f = jnp.min(jnp.where(m == mloc, mi, jnp.float32(99999.0)))
        li_b = jnp.broadcast_to(lidx_f.astype(jnp.int32), (_L,))
        cx0 = plsc.load_gather(x0_v, [li_b])
        cy0 = plsc.load_gather(y0_v, [li_b])
        cx1 = plsc.load_gather(x1_v, [li_b])
        cy1 = plsc.load_gather(y1_v, [li_b])
        row = jnp.broadcast_to(mloc, (_L,))
        row = jnp.where(lane == 1, lidx_f, row)
        row = jnp.where(lane == 2, cx0, row)
        row = jnp.where(lane == 3, cy0, row)
        row = jnp.where(lane == 4, cx1, row)
        row = jnp.where(lane == 5, cy1, row)
        cand_v[...] = row

    m = jnp.full((_L,), -2.0, jnp.float32)
    mi = jnp.zeros((_L,), jnp.float32)
    for j in range(_NV):
        v = sc_a[pl.ds(j * _L, _L)]
        gt = v > m
        mi = jnp.where(gt, jnp.float32(j * _L) + lane_f, mi)
        m = jnp.where(gt, v, m)
    publish_cand(m, mi)

    def substep(i, off, src, dst):
        pltpu.sync_copy(cand_v, shared_v.at[pl.ds(off + sid * _L, _L)])
        plsc.subcore_barrier()
        pltpu.sync_copy(shared_v.at[pl.ds(off, _MAT)], mat_v)

        maxes = plsc.load_gather(mat_v, [lane * _L])
        mg = jnp.max(maxes)
        wt = plsc.all_reduce_ffs(maxes == mg)
        wrow = wt * _L
        lidxw = plsc.load_gather(mat_v, [wrow + 1])
        wx0 = plsc.load_gather(mat_v, [wrow + 2])
        wy0 = plsc.load_gather(mat_v, [wrow + 3])
        wx1 = plsc.load_gather(mat_v, [wrow + 4])
        wy1 = plsc.load_gather(mat_v, [wrow + 5])
        warea = (wx1 - wx0) * (wy1 - wy0)

        mg_t = jnp.where(mg < jnp.float32(_SCORE_THRESH), jnp.float32(0.0),
                         mg)
        plsc.store_scatter(out_v, [jnp.broadcast_to(i, (_L,))],
                           jnp.broadcast_to(mg_t, (_L,)), mask=mask0)

        plsc.store_scatter(src, [lidxw.astype(jnp.int32)],
                           jnp.full((_L,), -1.0, jnp.float32),
                           mask=mask0 & (wt == sid))

        m0 = jnp.full((_L,), -2.0, jnp.float32)
        mi0 = jnp.zeros((_L,), jnp.float32)

        @plsc.parallel_loop(0, _CHUNK, _L, unroll=2, carry=(m0, mi0))
        def fused(iv, c):
            m, mi = c
            sl = pl.ds(iv, _L)
            iw = jnp.maximum(
                jnp.minimum(wx1, x1_v[sl]) - jnp.maximum(wx0, x0_v[sl]), 0.0)
            ih = jnp.maximum(
                jnp.minimum(wy1, y1_v[sl]) - jnp.maximum(wy0, y0_v[sl]), 0.0)
            inter = iw * ih
            union = warea + area_v[sl] - inter
            iou = inter / (union + jnp.float32(1e-9))
            dec = jnp.exp(-(iou * iou) / jnp.float32(_SIGMA))
            sc = src[sl] * dec
            dst[sl] = sc
            gt = sc > m
            mi = jnp.where(gt, iv.astype(jnp.float32) + lane_f, mi)
            m = jnp.where(gt, sc, m)
            return (m, mi)

        publish_cand(*fused)

    def step(k, carry):
        i = k * 2
        substep(i, 0, sc_a, sc_b)
        substep(i + 1, _MAT, sc_b, sc_a)
        return carry

    lax.fori_loop(0, _MAX_DET // 2, step, 0)

    @pl.when(jnp.logical_and(cid == 0, sid == 0))
    def _():
        pltpu.sync_copy(out_v.at[pl.ds(0, _MAX_DET)], out_hbm)


@jax.jit
def kernel(boxes, scores):
    x0 = jnp.zeros((_NPAD,), jnp.float32).at[:_N].set(boxes[:, 0])
    y0 = jnp.zeros((_NPAD,), jnp.float32).at[:_N].set(boxes[:, 1])
    x1 = jnp.zeros((_NPAD,), jnp.float32).at[:_N].set(boxes[:, 2])
    y1 = jnp.zeros((_NPAD,), jnp.float32).at[:_N].set(boxes[:, 3])
    sc = jnp.full((_NPAD,), -1.0, jnp.float32).at[:_N].set(scores)

    mesh = plsc.VectorSubcoreMesh(core_axis_name="c", subcore_axis_name="s",
                                  num_cores=2, num_subcores=_NS)
    run = pl.kernel(
        _nms_body,
        out_type=jax.ShapeDtypeStruct((_MAX_DET,), jnp.float32),
        mesh=mesh,
        scratch_types=[
            pltpu.VMEM((_CHUNK,), jnp.float32),
            pltpu.VMEM((_CHUNK,), jnp.float32),
            pltpu.VMEM((_CHUNK,), jnp.float32),
            pltpu.VMEM((_CHUNK,), jnp.float32),
            pltpu.VMEM((_CHUNK,), jnp.float32),
            pltpu.VMEM((_CHUNK,), jnp.float32),
            pltpu.VMEM((_CHUNK,), jnp.float32),
            pltpu.VMEM((_OUT_PAD,), jnp.float32),
            pltpu.VMEM((_L,), jnp.float32),
            pltpu.VMEM((_MAT,), jnp.float32),
            pltpu.VMEM_SHARED((2 * _MAT,), jnp.float32),
        ],
        compiler_params=pltpu.CompilerParams(needs_layout_passes=False),
    )
    return run(x0, y0, x1, y1, sc)


# speedup vs baseline: 1.3426x; 1.0094x over previous
import jax
import jax.numpy as jnp
from jax import lax
from jax.experimental import pallas as pl
from jax.experimental.pallas import tpu as pltpu
from jax.experimental.pallas import tpu_sc as plsc

_N = 5000
_MAX_DET = 300
_SIGMA = 0.5
_SCORE_THRESH = 0.001

_L = 16
_NS = 16
_NPAD = 5120
_CHUNK = _NPAD // _NS
_NV = _CHUNK // _L
_OUT_PAD = 320
_MAT = _NS * _L


def _nms_body(x0_hbm, y0_hbm, x1_hbm, y1_hbm, sc_hbm, out_hbm,
              x0_v, y0_v, x1_v, y1_v, area_v, sc_a, sc_b, out_v, cand_v,
              mat_v, shared_v):
    cid = lax.axis_index("c")
    sid = lax.axis_index("s")
    base = sid * _CHUNK

    pltpu.sync_copy(x0_hbm.at[pl.ds(base, _CHUNK)], x0_v)
    pltpu.sync_copy(y0_hbm.at[pl.ds(base, _CHUNK)], y0_v)
    pltpu.sync_copy(x1_hbm.at[pl.ds(base, _CHUNK)], x1_v)
    pltpu.sync_copy(y1_hbm.at[pl.ds(base, _CHUNK)], y1_v)
    pltpu.sync_copy(sc_hbm.at[pl.ds(base, _CHUNK)], sc_a)

    lane = lax.iota(jnp.int32, _L)
    lane_f = lane.astype(jnp.float32)
    mask0 = lane == 0

    for j in range(_NV):
        sl = pl.ds(j * _L, _L)
        area_v[sl] = (x1_v[sl] - x0_v[sl]) * (y1_v[sl] - y0_v[sl])

    def publish_cand(m, mi):
        mloc = jnp.max(m)
        lidx_f = jnp.min(jnp.where(m == mloc, mi, jnp.float32(99999.0)))
        li_b = jnp.broadcast_to(lidx_f.astype(jnp.int32), (_L,))
        cx0 = plsc.load_gather(x0_v, [li_b])
        cy0 = plsc.load_gather(y0_v, [li_b])
        cx1 = plsc.load_gather(x1_v, [li_b])
        cy1 = plsc.load_gather(y1_v, [li_b])
        row = jnp.broadcast_to(mloc, (_L,))
        row = jnp.where(lane == 1, lidx_f, row)
        row = jnp.where(lane == 2, cx0, row)
        row = jnp.where(lane == 3, cy0, row)
        row = jnp.where(lane == 4, cx1, row)
        row = jnp.where(lane == 5, cy1, row)
        cand_v[...] = row

    m = jnp.full((_L,), -2.0, jnp.float32)
    mi = jnp.zeros((_L,), jnp.float32)
    for j in range(_NV):
        v = sc_a[pl.ds(j * _L, _L)]
        gt = v > m
        mi = jnp.where(gt, jnp.float32(j * _L) + lane_f, mi)
        m = jnp.where(gt, v, m)
    publish_cand(m, mi)

    def substep(i, off, src, dst):
        pltpu.sync_copy(cand_v, shared_v.at[pl.ds(off + sid * _L, _L)])
        plsc.subcore_barrier()
        pltpu.sync_copy(shared_v.at[pl.ds(off, _MAT)], mat_v)

        maxes = plsc.load_gather(mat_v, [lane * _L])
        mg = jnp.max(maxes)
        wt = plsc.all_reduce_ffs(maxes == mg)
        wrow = wt * _L
        lidxw = plsc.load_gather(mat_v, [wrow + 1])
        wx0 = plsc.load_gather(mat_v, [wrow + 2])
        wy0 = plsc.load_gather(mat_v, [wrow + 3])
        wx1 = plsc.load_gather(mat_v, [wrow + 4])
        wy1 = plsc.load_gather(mat_v, [wrow + 5])
        warea = (wx1 - wx0) * (wy1 - wy0)

        mg_t = jnp.where(mg < jnp.float32(_SCORE_THRESH), jnp.float32(0.0),
                         mg)
        plsc.store_scatter(out_v, [jnp.broadcast_to(i, (_L,))],
                           jnp.broadcast_to(mg_t, (_L,)), mask=mask0)

        plsc.store_scatter(src, [lidxw.astype(jnp.int32)],
                           jnp.full((_L,), -1.0, jnp.float32),
                           mask=mask0 & (wt == sid))

        m0 = jnp.full((_L,), -2.0, jnp.float32)
        mi0 = jnp.zeros((_L,), jnp.float32)

        @plsc.parallel_loop(0, _CHUNK, _L, unroll=2, carry=(m0, mi0))
        def fused(iv, c):
            m, mi = c
            sl = pl.ds(iv, _L)
            iw = jnp.maximum(
                jnp.minimum(wx1, x1_v[sl]) - jnp.maximum(wx0, x0_v[sl]), 0.0)
            ih = jnp.maximum(
                jnp.minimum(wy1, y1_v[sl]) - jnp.maximum(wy0, y0_v[sl]), 0.0)
            inter = iw * ih
            union = warea + area_v[sl] - inter
            iou = inter / (union + jnp.float32(1e-9))
            dec = jnp.exp(-(iou * iou) / jnp.float32(_SIGMA))
            sc = src[sl] * dec
            dst[sl] = sc
            gt = sc > m
            mi = jnp.where(gt, iv.astype(jnp.float32) + lane_f, mi)
            m = jnp.where(gt, sc, m)
            return (m, mi)

        publish_cand(*fused)

    def step(k, carry):
        i = k * 2
        substep(i, 0, sc_a, sc_b)
        substep(i + 1, _MAT, sc_b, sc_a)
        return carry

    lax.fori_loop(0, _MAX_DET // 2, step, 0)

    @pl.when(jnp.logical_and(cid == 0, sid == 0))
    def _():
        pltpu.sync_copy(out_v.at[pl.ds(0, _MAX_DET)], out_hbm)


@jax.jit
def kernel(boxes, scores):
    x0 = jnp.zeros((_NPAD,), jnp.float32).at[:_N].set(boxes[:, 0])
    y0 = jnp.zeros((_NPAD,), jnp.float32).at[:_N].set(boxes[:, 1])
    x1 = jnp.zeros((_NPAD,), jnp.float32).at[:_N].set(boxes[:, 2])
    y1 = jnp.zeros((_NPAD,), jnp.float32).at[:_N].set(boxes[:, 3])
    sc = jnp.full((_NPAD,), -1.0, jnp.float32).at[:_N].set(scores)

    mesh = plsc.VectorSubcoreMesh(core_axis_name="c", subcore_axis_name="s",
                                  num_cores=1, num_subcores=_NS)
    run = pl.kernel(
        _nms_body,
        out_type=jax.ShapeDtypeStruct((_MAX_DET,), jnp.float32),
        mesh=mesh,
        scratch_types=[
            pltpu.VMEM((_CHUNK,), jnp.float32),
            pltpu.VMEM((_CHUNK,), jnp.float32),
            pltpu.VMEM((_CHUNK,), jnp.float32),
            pltpu.VMEM((_CHUNK,), jnp.float32),
            pltpu.VMEM((_CHUNK,), jnp.float32),
            pltpu.VMEM((_CHUNK,), jnp.float32),
            pltpu.VMEM((_CHUNK,), jnp.float32),
            pltpu.VMEM((_OUT_PAD,), jnp.float32),
            pltpu.VMEM((_L,), jnp.float32),
            pltpu.VMEM((_MAT,), jnp.float32),
            pltpu.VMEM_SHARED((2 * _MAT,), jnp.float32),
        ],
        compiler_params=pltpu.CompilerParams(needs_layout_passes=False),
    )
    return run(x0, y0, x1, y1, sc)
